# baseline (speedup 1.0000x reference)
import jax
import jax.numpy as jnp
from jax import lax
from jax.experimental import pallas as pl
from jax.experimental.pallas import tpu as pltpu
from jax.experimental.pallas import tpu_sc as plsc

B, S, D = 4, 8192, 768
NC, NS = 2, 16
NW = NC * NS
SPW = S // NW
SCHUNK = 8
NSC = SPW // SCHUNK
NTASK = NSC * B
LANES = 16
NBUF = 12


def _sc_body(x_hbm, pos_hbm, out_hbm, *scr):
    wid = lax.axis_index("s") * NC + lax.axis_index("c")
    s0 = wid * SPW
    posb = scr[0:2]
    xb = scr[2:2 + NBUF]
    sems = scr[2 + NBUF:]
    sp = sems[0:2]
    si = sems[2:2 + NBUF]
    so = sems[2 + NBUF:2 + 2 * NBUF]

    def xrow(t):
        sc, b = divmod(t, B)
        return b * S + s0 + sc * SCHUNK

    def pos_slice(sc):
        return pos_hbm.at[pl.ds(s0 + sc * SCHUNK, SCHUNK)]

    pin = [pltpu.async_copy(pos_slice(0), posb[0], sp[0]),
           pltpu.async_copy(pos_slice(1), posb[1], sp[1])]
    xin = [None] * NBUF
    xout = [None] * NBUF
    xin[0] = pltpu.async_copy(x_hbm.at[pl.ds(xrow(0), SCHUNK)], xb[0], si[0])
    xin[1] = pltpu.async_copy(x_hbm.at[pl.ds(xrow(1), SCHUNK)], xb[1], si[1])

    for t in range(NTASK):
        sc, b = divmod(t, B)
        p = t % NBUF
        pc = sc & 1
        if b == 0:
            pin[pc].wait()
        xin[p].wait()
        if t >= 1:
            q = (t - 1) % NBUF
            xout[q] = pltpu.async_copy(
                xb[q], out_hbm.at[pl.ds(xrow(t - 1), SCHUNK)], so[q])
        if t + 2 < NTASK:
            r = (t + 2) % NBUF
            if xout[r] is not None:
                xout[r].wait()
            xin[r] = pltpu.async_copy(
                x_hbm.at[pl.ds(xrow(t + 2), SCHUNK)], xb[r], si[r])

        @plsc.parallel_loop(0, SCHUNK, step=1)
        def _(rr):
            @plsc.parallel_loop(0, D, step=LANES, unroll=8)
            def _(c):
                plsc.addupdate(xb[p].at[rr].at[pl.ds(c, LANES)],
                               posb[pc].at[rr][pl.ds(c, LANES)])

        if b == B - 1 and sc + 2 < NSC:
            pin[pc] = pltpu.async_copy(pos_slice(sc + 2), posb[pc], sp[pc])

    last = (NTASK - 1) % NBUF
    xout[last] = pltpu.async_copy(
        xb[last], out_hbm.at[pl.ds(xrow(NTASK - 1), SCHUNK)], so[last])
    for q in range(NBUF):
        if xout[q] is not None:
            xout[q].wait()


_sc_call = pl.kernel(
    _sc_body,
    out_type=jax.ShapeDtypeStruct((B * S, D), jnp.float32),
    mesh=plsc.VectorSubcoreMesh(core_axis_name="c", subcore_axis_name="s"),
    scratch_types=(
        [pltpu.VMEM((SCHUNK, D), jnp.float32)] * 2
        + [pltpu.VMEM((SCHUNK, D), jnp.float32)] * NBUF
        + [pltpu.SemaphoreType.DMA] * (2 + 2 * NBUF)
    ),
)


def kernel(x, pos_table):
    out = _sc_call(x.reshape(B * S, D), pos_table)
    return out.reshape(B, S, D)
